# baseline (speedup 1.0000x reference)
import functools
import jax
import jax.numpy as jnp
from jax import lax
from jax.experimental import pallas as pl
from jax.experimental.pallas import tpu as pltpu
from jax.experimental.pallas import tpu_sc as plsc

B, N, D, K = 4, 2048, 128, 24
BN = B * N
TOK = BN * K

BIG_F = 3.0e38
BIG_I = 2**30

QBLK = 128
CHR = 256
NCH = N // CHR


KH = 12


def _topk_body(h_all_ref, hq_ref, idx_ref, score_ref, cv_ref, ci_ref, m16_ref):
    b = pl.program_id(0)
    j = pl.program_id(2)

    @pl.when(j == 0)
    def _scores():
        h_all = h_all_ref[0]
        sq = jnp.sum(h_all * h_all, axis=1, keepdims=True) * 0.5
        dot = lax.dot_general(h_all, hq_ref[0], (((1,), (0,)), ((), ())),
                              preferred_element_type=jnp.float32)
        score = sq - dot
        score_ref[...] = score
        m16_ref[...] = jnp.min(score.reshape(N // 16, 16, QBLK), axis=1)

    @pl.when(j == 1)
    def _extract():
        kiota = lax.broadcasted_iota(jnp.int32, (K, QBLK), 0)
        riota = lax.broadcasted_iota(jnp.int32, (CHR, QBLK), 0)

        def ext_tau(k, carry):
            mv16, _ = carry
            m = jnp.min(mv16, axis=0, keepdims=True)
            mv16 = jnp.where(mv16 == m, BIG_F, mv16)
            return mv16, m

        _, tau = lax.fori_loop(
            0, K, ext_tau, (m16_ref[...], jnp.zeros((1, QBLK), jnp.float32)))

        def ext_local(k, carry):
            ch, mv, mi = carry
            m = jnp.min(ch, axis=0, keepdims=True)
            key = jnp.where(ch == m, riota, BIG_I)
            am = jnp.min(key, axis=0, keepdims=True)
            ch = jnp.where(key == am, BIG_F, ch)
            mv = jnp.where(kiota == k, m, mv)
            mi = jnp.where(kiota == k, am, mi)
            return ch, mv, mi

        for c in range(NCH):
            ch0 = score_ref[c * CHR:(c + 1) * CHR, :]
            ch1, mv, mi = lax.fori_loop(
                0, KH, ext_local,
                (ch0, jnp.full((K, QBLK), BIG_F, jnp.float32),
                 jnp.zeros((K, QBLK), jnp.int32)))
            cv_ref[c * K:(c + 1) * K, :] = mv
            ci_ref[c * K:(c + 1) * K, :] = mi + c * CHR
            mnext = jnp.min(ch1, axis=0, keepdims=True)
            need_more = jnp.any(mnext <= tau)

            @pl.when(need_more)
            def _finish(ch1=ch1, mv=mv, mi=mi, c=c):
                _, mv2, mi2 = lax.fori_loop(KH, K, ext_local, (ch1, mv, mi))
                cv_ref[c * K:(c + 1) * K, :] = mv2
                ci_ref[c * K:(c + 1) * K, :] = mi2 + c * CHR

        v0 = cv_ref[...]
        ix = ci_ref[...]

        def ext_merge(k, carry):
            v, cols = carry
            m = jnp.min(v, axis=0, keepdims=True)
            key = jnp.where(v == m, ix, BIG_I)
            am = jnp.min(key, axis=0, keepdims=True)
            v = jnp.where(key == am, BIG_F, v)
            cols = jnp.where(kiota == k, am, cols)
            return v, cols

        _, cols = lax.fori_loop(
            0, K, ext_merge, (v0, jnp.zeros((K, QBLK), jnp.int32)))
        idx_ref[...] = cols + b * N


def _topk(h):
    return pl.pallas_call(
        _topk_body,
        grid=(B, N // QBLK, 2),
        in_specs=[
            pl.BlockSpec((1, N, D), lambda b, i, j: (b, 0, 0)),
            pl.BlockSpec((1, D, QBLK), lambda b, i, j: (b, 0, i)),
        ],
        out_specs=pl.BlockSpec(
            (K, QBLK), lambda b, i, j: (0, b * (N // QBLK) + i)),
        out_shape=jax.ShapeDtypeStruct((K, BN), jnp.int32),
        scratch_shapes=[pltpu.VMEM((N, QBLK), jnp.float32),
                        pltpu.VMEM((NCH * K, QBLK), jnp.float32),
                        pltpu.VMEM((NCH * K, QBLK), jnp.int32),
                        pltpu.VMEM((N // 16, QBLK), jnp.float32)],
    )(h, jnp.swapaxes(h, 1, 2))


PBLK = 512


def _pre_body(h_ref, rf_ref, p_ref, w1hi_ref, w1hj_ref, w1p_ref, b1_ref,
              prej_ref, base_ref, g_ref):
    h = h_ref[...]
    rf = rf_ref[...]
    pp = p_ref[...]
    w1p = w1p_ref[...]
    prej_ref[...] = lax.dot_general(h, w1hj_ref[...], (((1,), (0,)), ((), ())),
                                    preferred_element_type=jnp.float32)
    prei = lax.dot_general(h, w1hi_ref[...], (((1,), (0,)), ((), ())),
                           preferred_element_type=jnp.float32)
    gs = []
    pterm = jnp.zeros((PBLK, D), jnp.float32)
    for c in range(3):
        g_c = (rf[:, 3 * c + 0:3 * c + 1] * w1p[0:1, :]
               + rf[:, 3 * c + 1:3 * c + 2] * w1p[1:2, :]
               + rf[:, 3 * c + 2:3 * c + 3] * w1p[2:3, :])
        gs.append(g_c)
        pterm = pterm + pp[:, c:c + 1] * g_c
    g_ref[...] = jnp.concatenate(gs, axis=1)
    base_ref[...] = prei + b1_ref[...] - pterm


def _precompute(hf, rf, pf, w1hi, w1hj, w1p, b1):
    return pl.pallas_call(
        _pre_body,
        grid=(BN // PBLK,),
        in_specs=[
            pl.BlockSpec((PBLK, D), lambda i: (i, 0)),
            pl.BlockSpec((PBLK, 16), lambda i: (i, 0)),
            pl.BlockSpec((PBLK, 8), lambda i: (i, 0)),
            pl.BlockSpec((D, D), lambda i: (0, 0)),
            pl.BlockSpec((D, D), lambda i: (0, 0)),
            pl.BlockSpec((8, D), lambda i: (0, 0)),
            pl.BlockSpec((1, D), lambda i: (0, 0)),
        ],
        out_specs=[
            pl.BlockSpec((PBLK, D), lambda i: (i, 0)),
            pl.BlockSpec((PBLK, D), lambda i: (i, 0)),
            pl.BlockSpec((PBLK, 3 * D), lambda i: (i, 0)),
        ],
        out_shape=[
            jax.ShapeDtypeStruct((BN, D), jnp.float32),
            jax.ShapeDtypeStruct((BN, D), jnp.float32),
            jax.ShapeDtypeStruct((BN, 3 * D), jnp.float32),
        ],
    )(hf, rf, pf, w1hi, w1hj, w1p, b1)


CQ = 64
CT = CQ * K


def _mlp_body(prej_ref, pj0_ref, pj1_ref, pj2_ref, base_ref, g_ref, w2_ref,
              b2_ref, w3_ref, b3_ref, out_ref):
    prej = prej_ref[...]
    base = base_ref[...]
    g = g_ref[...]
    t = prej.reshape(CQ, K, D) + base[:, None, :]
    for c, pj_ref in enumerate((pj0_ref, pj1_ref, pj2_ref)):
        t = t + pj_ref[...][:, :, None] * g[:, c * D:(c + 1) * D][:, None, :]
    x = jnp.where(t > 0, t, 0.1 * t).reshape(CT, D)
    x = lax.dot_general(x, w2_ref[...], (((1,), (0,)), ((), ())),
                        preferred_element_type=jnp.float32) + b2_ref[...]
    x = jnp.where(x > 0, x, 0.1 * x)
    x = lax.dot_general(x, w3_ref[...], (((1,), (0,)), ((), ())),
                        preferred_element_type=jnp.float32) + b3_ref[...]
    out_ref[...] = jnp.max(x.reshape(CQ, K, D), axis=1)


BN2 = BN // 2


def _mlp_call(prej_g, pj0, pj1, pj2, base, g, w2, b2, w3, b3):
    return pl.pallas_call(
        _mlp_body,
        grid=(BN2 // CQ,),
        in_specs=[
            pl.BlockSpec((CT, D), lambda i: (i, 0)),
            pl.BlockSpec((CQ, K), lambda i: (i, 0)),
            pl.BlockSpec((CQ, K), lambda i: (i, 0)),
            pl.BlockSpec((CQ, K), lambda i: (i, 0)),
            pl.BlockSpec((CQ, D), lambda i: (i, 0)),
            pl.BlockSpec((CQ, 3 * D), lambda i: (i, 0)),
            pl.BlockSpec((D, D), lambda i: (0, 0)),
            pl.BlockSpec((1, D), lambda i: (0, 0)),
            pl.BlockSpec((D, D), lambda i: (0, 0)),
            pl.BlockSpec((1, D), lambda i: (0, 0)),
        ],
        out_specs=pl.BlockSpec((CQ, D), lambda i: (i, 0)),
        out_shape=jax.ShapeDtypeStruct((BN2, D), jnp.float32),
    )(prej_g, pj0, pj1, pj2, base, g, w2, b2, w3, b3)


NWORK = 32
TOK2 = TOK // 2
GW = TOK2 // NWORK
GC = 128
NWIN = GW // GC


def _gather(prej_table, ptab_flat, idx_flat):
    mesh = plsc.VectorSubcoreMesh(core_axis_name="c", subcore_axis_name="s")

    @functools.partial(
        pl.kernel, mesh=mesh,
        out_type=[jax.ShapeDtypeStruct((TOK2, D), jnp.float32),
                  jax.ShapeDtypeStruct((3 * TOK2,), jnp.float32)],
        scratch_types=[pltpu.VMEM((GW,), jnp.int32),
                       pltpu.VMEM((GC, D), jnp.float32),
                       pltpu.VMEM((GC, D), jnp.float32),
                       pltpu.VMEM((3 * GC,), jnp.int32),
                       pltpu.VMEM((3 * GC,), jnp.int32),
                       pltpu.VMEM((3 * GC,), jnp.float32),
                       pltpu.VMEM((3 * GC,), jnp.float32),
                       pltpu.SemaphoreType.DMA,
                       pltpu.SemaphoreType.DMA,
                       pltpu.SemaphoreType.DMA,
                       pltpu.SemaphoreType.DMA],
    )
    def k(prej_hbm, ptab_hbm, idx_hbm, o1_hbm, o2_hbm,
          idx_v, rows_a, rows_b, offs_a, offs_b, pv_a, pv_b, sa, sb, pa, pb):
        wid = lax.axis_index("s") * 2 + lax.axis_index("c")
        base = wid * GW
        pltpu.sync_copy(idx_hbm.at[pl.ds(base, GW)], idx_v)

        def fill_offs(w, offs):
            for g in range(GC // 16):
                iv = idx_v[pl.ds(w * GC + g * 16, 16)]
                for c in range(3):
                    offs[pl.ds(c * GC + g * 16, 16)] = iv * 4 + c

        def start_win(w, offs, rows, pv, sr, sp):
            fill_offs(w, offs)
            hr = pltpu.async_copy(
                prej_hbm.at[idx_v.at[pl.ds(w * GC, GC)]], rows, sr)
            hp = pltpu.async_copy(ptab_hbm.at[offs], pv, sp)
            return hr, hp

        def finish_win(w, rows, pv, hr, hp):
            hr.wait()
            pltpu.sync_copy(rows, o1_hbm.at[pl.ds(base + w * GC, GC)])
            hp.wait()
            for c in range(3):
                pltpu.sync_copy(
                    pv.at[pl.ds(c * GC, GC)],
                    o2_hbm.at[pl.ds(c * TOK2 + base + w * GC, GC)])

        def pair(g, _):
            wa = 2 * g
            wb = 2 * g + 1
            ha = start_win(wa, offs_a, rows_a, pv_a, sa, pa)
            hb = start_win(wb, offs_b, rows_b, pv_b, sb, pb)
            finish_win(wa, rows_a, pv_a, *ha)
            finish_win(wb, rows_b, pv_b, *hb)
            return 0

        lax.fori_loop(0, NWIN // 2, pair, 0)

    return k(prej_table, ptab_flat, idx_flat)


@jax.jit
def kernel(p, R, h, W1, b1, W2, b2, W3, b3):
    hf = h.reshape(BN, D)
    rf = jnp.pad(R.reshape(BN, 9), ((0, 0), (0, 7)))
    pf = jnp.pad(p.reshape(BN, 3), ((0, 0), (0, 5)))
    w1p = jnp.pad(W1[0:3], ((0, 5), (0, 0)))
    w1hi = W1[3:3 + D]
    w1hj = W1[3 + D:3 + 2 * D]
    b1r = b1.reshape(1, D)

    idx = _topk(h).T
    prej_table, base, g = _precompute(hf, rf, pf, w1hi, w1hj, w1p, b1r)
    ptab = jnp.pad(p.reshape(BN, 3), ((0, 0), (0, 1))).reshape(BN * 4)

    idxf = idx.reshape(TOK)
    b2r = b2.reshape(1, D)
    b3r = b3.reshape(1, D)
    outs = []
    for hh in range(2):
        prej_g, pj_flat = _gather(prej_table, ptab,
                                  idxf[hh * TOK2:(hh + 1) * TOK2])
        pj3 = pj_flat.reshape(3, BN2, K)
        outs.append(_mlp_call(prej_g, pj3[0], pj3[1], pj3[2],
                              base[hh * BN2:(hh + 1) * BN2],
                              g[hh * BN2:(hh + 1) * BN2],
                              W2, b2r, W3, b3r))
    return jnp.concatenate(outs, axis=0).reshape(B, N, D)


# speedup vs baseline: 1.1159x; 1.1159x over previous
import functools
import jax
import jax.numpy as jnp
from jax import lax
from jax.experimental import pallas as pl
from jax.experimental.pallas import tpu as pltpu
from jax.experimental.pallas import tpu_sc as plsc

B, N, D, K = 4, 2048, 128, 24
BN = B * N
TOK = BN * K

BIG_F = 3.0e38
BIG_I = 2**30

QBLK = 128
QB2 = 256
CHR = 256
NCH = N // CHR


KH = 10


def _topk_body(h_all_ref, hq_ref, idx_ref, score_ref, cv_ref, ci_ref, m16_ref):
    b = pl.program_id(0)
    j = pl.program_id(2)

    @pl.when(j == 0)
    def _scores():
        h_all = h_all_ref[0]
        sq = jnp.sum(h_all * h_all, axis=1, keepdims=True) * 0.5
        dot = lax.dot_general(h_all, hq_ref[0], (((1,), (0,)), ((), ())),
                              preferred_element_type=jnp.float32)
        score = sq - dot
        score_ref[...] = score
        m16_ref[...] = jnp.min(score.reshape(N // 16, 16, QB2), axis=1)

    def _extract(hh):
        q0 = hh * QBLK
        kiota = lax.broadcasted_iota(jnp.int32, (K, QBLK), 0)
        riota = lax.broadcasted_iota(jnp.int32, (CHR, QBLK), 0)

        def ext_tau(k, carry):
            mv16, _ = carry
            m = jnp.min(mv16, axis=0, keepdims=True)
            mv16 = jnp.where(mv16 == m, BIG_F, mv16)
            return mv16, m

        _, tau = lax.fori_loop(
            0, K, ext_tau, (m16_ref[:, q0:q0 + QBLK],
                            jnp.zeros((1, QBLK), jnp.float32)))

        def ext_local(k, carry):
            ch, mv, mi = carry
            m = jnp.min(ch, axis=0, keepdims=True)
            key = jnp.where(ch == m, riota, BIG_I)
            am = jnp.min(key, axis=0, keepdims=True)
            ch = jnp.where(key == am, BIG_F, ch)
            mv = jnp.where(kiota == k, m, mv)
            mi = jnp.where(kiota == k, am, mi)
            return ch, mv, mi

        for c in range(NCH):
            ch0 = score_ref[c * CHR:(c + 1) * CHR, q0:q0 + QBLK]
            ch1, mv, mi = lax.fori_loop(
                0, KH, ext_local,
                (ch0, jnp.full((K, QBLK), BIG_F, jnp.float32),
                 jnp.zeros((K, QBLK), jnp.int32)))
            cv_ref[c * K:(c + 1) * K, :] = mv
            ci_ref[c * K:(c + 1) * K, :] = mi + c * CHR
            mnext = jnp.min(ch1, axis=0, keepdims=True)
            need_more = jnp.any(mnext <= tau)

            @pl.when(need_more)
            def _finish(ch1=ch1, mv=mv, mi=mi, c=c):
                _, mv2, mi2 = lax.fori_loop(KH, K, ext_local, (ch1, mv, mi))
                cv_ref[c * K:(c + 1) * K, :] = mv2
                ci_ref[c * K:(c + 1) * K, :] = mi2 + c * CHR

        v0 = cv_ref[...]
        ix = ci_ref[...]

        def ext_merge(k, carry):
            v, cols = carry
            m = jnp.min(v, axis=0, keepdims=True)
            key = jnp.where(v == m, ix, BIG_I)
            am = jnp.min(key, axis=0, keepdims=True)
            v = jnp.where(key == am, BIG_F, v)
            cols = jnp.where(kiota == k, am, cols)
            return v, cols

        _, cols = lax.fori_loop(
            0, K, ext_merge, (v0, jnp.zeros((K, QBLK), jnp.int32)))
        idx_ref[...] = cols + b * N

    for _hh in range(2):
        @pl.when(j == 1 + _hh)
        def _run(_hh=_hh):
            _extract(_hh)


def _topk(h):
    return pl.pallas_call(
        _topk_body,
        grid=(B, N // QB2, 3),
        in_specs=[
            pl.BlockSpec((1, N, D), lambda b, i, j: (b, 0, 0)),
            pl.BlockSpec((1, D, QB2), lambda b, i, j: (b, 0, i)),
        ],
        out_specs=pl.BlockSpec(
            (K, QBLK),
            lambda b, i, j: (0, (b * (N // QB2) + i) * 2 +
                             jnp.maximum(j - 1, 0))),
        out_shape=jax.ShapeDtypeStruct((K, BN), jnp.int32),
        scratch_shapes=[pltpu.VMEM((N, QB2), jnp.float32),
                        pltpu.VMEM((NCH * K, QBLK), jnp.float32),
                        pltpu.VMEM((NCH * K, QBLK), jnp.int32),
                        pltpu.VMEM((N // 16, QB2), jnp.float32)],
    )(h, jnp.swapaxes(h, 1, 2))


PBLK = 512


def _pre_body(h_ref, rf_ref, p_ref, w1hi_ref, w1hj_ref, w1p_ref, b1_ref,
              prej_ref, base_ref, g_ref):
    h = h_ref[...]
    rf = rf_ref[...]
    pp = p_ref[...]
    w1p = w1p_ref[...]
    prej_ref[...] = lax.dot_general(h, w1hj_ref[...], (((1,), (0,)), ((), ())),
                                    preferred_element_type=jnp.float32)
    prei = lax.dot_general(h, w1hi_ref[...], (((1,), (0,)), ((), ())),
                           preferred_element_type=jnp.float32)
    gs = []
    pterm = jnp.zeros((PBLK, D), jnp.float32)
    for c in range(3):
        g_c = (rf[:, 3 * c + 0:3 * c + 1] * w1p[0:1, :]
               + rf[:, 3 * c + 1:3 * c + 2] * w1p[1:2, :]
               + rf[:, 3 * c + 2:3 * c + 3] * w1p[2:3, :])
        gs.append(g_c)
        pterm = pterm + pp[:, c:c + 1] * g_c
    g_ref[...] = jnp.concatenate(gs, axis=1)
    base_ref[...] = prei + b1_ref[...] - pterm


def _precompute(hf, rf, pf, w1hi, w1hj, w1p, b1):
    return pl.pallas_call(
        _pre_body,
        grid=(BN // PBLK,),
        in_specs=[
            pl.BlockSpec((PBLK, D), lambda i: (i, 0)),
            pl.BlockSpec((PBLK, 16), lambda i: (i, 0)),
            pl.BlockSpec((PBLK, 8), lambda i: (i, 0)),
            pl.BlockSpec((D, D), lambda i: (0, 0)),
            pl.BlockSpec((D, D), lambda i: (0, 0)),
            pl.BlockSpec((8, D), lambda i: (0, 0)),
            pl.BlockSpec((1, D), lambda i: (0, 0)),
        ],
        out_specs=[
            pl.BlockSpec((PBLK, D), lambda i: (i, 0)),
            pl.BlockSpec((PBLK, D), lambda i: (i, 0)),
            pl.BlockSpec((PBLK, 3 * D), lambda i: (i, 0)),
        ],
        out_shape=[
            jax.ShapeDtypeStruct((BN, D), jnp.float32),
            jax.ShapeDtypeStruct((BN, D), jnp.float32),
            jax.ShapeDtypeStruct((BN, 3 * D), jnp.float32),
        ],
    )(hf, rf, pf, w1hi, w1hj, w1p, b1)


CQ = 64
CT = CQ * K


def _mlp_body(prej_ref, pj0_ref, pj1_ref, pj2_ref, base_ref, g_ref, w2_ref,
              b2_ref, w3_ref, b3_ref, out_ref):
    prej = prej_ref[...]
    base = base_ref[...]
    g = g_ref[...]
    t = prej.reshape(CQ, K, D) + base[:, None, :]
    for c, pj_ref in enumerate((pj0_ref, pj1_ref, pj2_ref)):
        t = t + pj_ref[...][:, :, None] * g[:, c * D:(c + 1) * D][:, None, :]
    x = jnp.where(t > 0, t, 0.1 * t).reshape(CT, D)
    x = lax.dot_general(x, w2_ref[...], (((1,), (0,)), ((), ())),
                        preferred_element_type=jnp.float32) + b2_ref[...]
    x = jnp.where(x > 0, x, 0.1 * x)
    x = lax.dot_general(x, w3_ref[...], (((1,), (0,)), ((), ())),
                        preferred_element_type=jnp.float32) + b3_ref[...]
    out_ref[...] = jnp.max(x.reshape(CQ, K, D), axis=1)


def _mlp_call(prej_g, pj0, pj1, pj2, base, g, w2, b2, w3, b3):
    return pl.pallas_call(
        _mlp_body,
        grid=(BN // CQ,),
        in_specs=[
            pl.BlockSpec((CT, D), lambda i: (i, 0)),
            pl.BlockSpec((CQ, K), lambda i: (i, 0)),
            pl.BlockSpec((CQ, K), lambda i: (i, 0)),
            pl.BlockSpec((CQ, K), lambda i: (i, 0)),
            pl.BlockSpec((CQ, D), lambda i: (i, 0)),
            pl.BlockSpec((CQ, 3 * D), lambda i: (i, 0)),
            pl.BlockSpec((D, D), lambda i: (0, 0)),
            pl.BlockSpec((1, D), lambda i: (0, 0)),
            pl.BlockSpec((D, D), lambda i: (0, 0)),
            pl.BlockSpec((1, D), lambda i: (0, 0)),
        ],
        out_specs=pl.BlockSpec((CQ, D), lambda i: (i, 0)),
        out_shape=jax.ShapeDtypeStruct((BN, D), jnp.float32),
    )(prej_g, pj0, pj1, pj2, base, g, w2, b2, w3, b3)


NWORK = 32
GW = TOK // NWORK
GC = 128
NWIN = GW // GC


def _gather(prej_table, ptab_flat, idx_flat):
    mesh = plsc.VectorSubcoreMesh(core_axis_name="c", subcore_axis_name="s")

    @functools.partial(
        pl.kernel, mesh=mesh,
        out_type=[jax.ShapeDtypeStruct((TOK, D), jnp.float32),
                  jax.ShapeDtypeStruct((3 * TOK,), jnp.float32)],
        scratch_types=[pltpu.VMEM((GW,), jnp.int32),
                       pltpu.VMEM((GC, D), jnp.float32),
                       pltpu.VMEM((GC, D), jnp.float32),
                       pltpu.VMEM((3 * GC,), jnp.int32),
                       pltpu.VMEM((3 * GC,), jnp.int32),
                       pltpu.VMEM((3 * GC,), jnp.float32),
                       pltpu.VMEM((3 * GC,), jnp.float32),
                       pltpu.SemaphoreType.DMA,
                       pltpu.SemaphoreType.DMA,
                       pltpu.SemaphoreType.DMA,
                       pltpu.SemaphoreType.DMA],
    )
    def k(prej_hbm, ptab_hbm, idx_hbm, o1_hbm, o2_hbm,
          idx_v, rows_a, rows_b, offs_a, offs_b, pv_a, pv_b, sa, sb, pa, pb):
        wid = lax.axis_index("s") * 2 + lax.axis_index("c")
        base = wid * GW
        pltpu.sync_copy(idx_hbm.at[pl.ds(base, GW)], idx_v)

        def fill_offs(w, offs):
            for g in range(GC // 16):
                iv = idx_v[pl.ds(w * GC + g * 16, 16)]
                for c in range(3):
                    offs[pl.ds(c * GC + g * 16, 16)] = iv * 4 + c

        def start_win(w, offs, rows, pv, sr, sp):
            fill_offs(w, offs)
            hr = pltpu.async_copy(
                prej_hbm.at[idx_v.at[pl.ds(w * GC, GC)]], rows, sr)
            hp = pltpu.async_copy(ptab_hbm.at[offs], pv, sp)
            return hr, hp

        def finish_win(w, rows, pv, hr, hp):
            hr.wait()
            pltpu.sync_copy(rows, o1_hbm.at[pl.ds(base + w * GC, GC)])
            hp.wait()
            for c in range(3):
                pltpu.sync_copy(
                    pv.at[pl.ds(c * GC, GC)],
                    o2_hbm.at[pl.ds(c * TOK + base + w * GC, GC)])

        def pair(g, _):
            wa = 2 * g
            wb = 2 * g + 1
            ha = start_win(wa, offs_a, rows_a, pv_a, sa, pa)
            hb = start_win(wb, offs_b, rows_b, pv_b, sb, pb)
            finish_win(wa, rows_a, pv_a, *ha)
            finish_win(wb, rows_b, pv_b, *hb)
            return 0

        lax.fori_loop(0, NWIN // 2, pair, 0)

    return k(prej_table, ptab_flat, idx_flat)


@jax.jit
def kernel(p, R, h, W1, b1, W2, b2, W3, b3):
    hf = h.reshape(BN, D)
    rf = jnp.pad(R.reshape(BN, 9), ((0, 0), (0, 7)))
    pf = jnp.pad(p.reshape(BN, 3), ((0, 0), (0, 5)))
    w1p = jnp.pad(W1[0:3], ((0, 5), (0, 0)))
    w1hi = W1[3:3 + D]
    w1hj = W1[3 + D:3 + 2 * D]
    b1r = b1.reshape(1, D)

    idx = _topk(h).T
    prej_table, base, g = _precompute(hf, rf, pf, w1hi, w1hj, w1p, b1r)
    ptab = jnp.pad(p.reshape(BN, 3), ((0, 0), (0, 1))).reshape(BN * 4)

    prej_g, pj_flat = _gather(prej_table, ptab, idx.reshape(TOK))
    pj3 = pj_flat.reshape(3, BN, K)
    out = _mlp_call(prej_g, pj3[0], pj3[1], pj3[2], base, g, W2,
                    b2.reshape(1, D), W3, b3.reshape(1, D))
    return out.reshape(B, N, D)


# speedup vs baseline: 1.1572x; 1.0370x over previous
import functools
import jax
import jax.numpy as jnp
from jax import lax
from jax.experimental import pallas as pl
from jax.experimental.pallas import tpu as pltpu
from jax.experimental.pallas import tpu_sc as plsc

B, N, D, K = 4, 2048, 128, 24
BN = B * N
TOK = BN * K

BIG_F = 3.0e38
BIG_I = 2**30

QBLK = 128
QB2 = 256
CHR = 256
NCH = N // CHR


KH = 10


def _topk_body(h_all_ref, hq_ref, idx_ref, score_ref, cv_ref, ci_ref, m16_ref):
    b = pl.program_id(0)
    j = pl.program_id(2)

    @pl.when(j == 0)
    def _scores():
        h_all = h_all_ref[0]
        sq = jnp.sum(h_all * h_all, axis=1, keepdims=True) * 0.5
        dot = lax.dot_general(h_all, hq_ref[0], (((1,), (0,)), ((), ())),
                              preferred_element_type=jnp.float32)
        score = sq - dot
        score_ref[...] = score
        m16_ref[...] = jnp.min(score.reshape(N // 16, 16, QB2), axis=1)

    def _extract(hh):
        q0 = hh * QBLK
        kiota = lax.broadcasted_iota(jnp.int32, (K, QBLK), 0)
        riota = lax.broadcasted_iota(jnp.int32, (CHR, QBLK), 0)

        def ext_tau(k, carry):
            mv16, _ = carry
            m = jnp.min(mv16, axis=0, keepdims=True)
            mv16 = jnp.where(mv16 == m, BIG_F, mv16)
            return mv16, m

        _, tau = lax.fori_loop(
            0, K, ext_tau, (m16_ref[:, q0:q0 + QBLK],
                            jnp.zeros((1, QBLK), jnp.float32)))

        def ext_local(k, carry):
            ch, mv, mi = carry
            m = jnp.min(ch, axis=0, keepdims=True)
            key = jnp.where(ch == m, riota, BIG_I)
            am = jnp.min(key, axis=0, keepdims=True)
            ch = jnp.where(key == am, BIG_F, ch)
            mv = jnp.where(kiota == k, m, mv)
            mi = jnp.where(kiota == k, am, mi)
            return ch, mv, mi

        for c in range(NCH):
            ch0 = score_ref[c * CHR:(c + 1) * CHR, q0:q0 + QBLK]
            ch1, mv, mi = lax.fori_loop(
                0, KH, ext_local,
                (ch0, jnp.full((K, QBLK), BIG_F, jnp.float32),
                 jnp.zeros((K, QBLK), jnp.int32)))
            cv_ref[c * K:(c + 1) * K, :] = mv
            ci_ref[c * K:(c + 1) * K, :] = mi + c * CHR
            mnext = jnp.min(ch1, axis=0, keepdims=True)
            need_more = jnp.any(mnext <= tau)

            @pl.when(need_more)
            def _finish(ch1=ch1, mv=mv, mi=mi, c=c):
                _, mv2, mi2 = lax.fori_loop(KH, K, ext_local, (ch1, mv, mi))
                cv_ref[c * K:(c + 1) * K, :] = mv2
                ci_ref[c * K:(c + 1) * K, :] = mi2 + c * CHR

        v0 = cv_ref[...]
        ix = ci_ref[...]

        def ext_merge(k, carry):
            v, cols = carry
            m = jnp.min(v, axis=0, keepdims=True)
            key = jnp.where(v == m, ix, BIG_I)
            am = jnp.min(key, axis=0, keepdims=True)
            v = jnp.where(key == am, BIG_F, v)
            cols = jnp.where(kiota == k, am, cols)
            return v, cols

        _, cols = lax.fori_loop(
            0, K, ext_merge, (v0, jnp.zeros((K, QBLK), jnp.int32)))
        idx_ref[...] = cols + b * N

    for _hh in range(2):
        @pl.when(j == 1 + _hh)
        def _run(_hh=_hh):
            _extract(_hh)


def _topk(h):
    return pl.pallas_call(
        _topk_body,
        grid=(B, N // QB2, 3),
        in_specs=[
            pl.BlockSpec((1, N, D), lambda b, i, j: (b, 0, 0)),
            pl.BlockSpec((1, D, QB2), lambda b, i, j: (b, 0, i)),
        ],
        out_specs=pl.BlockSpec(
            (K, QBLK),
            lambda b, i, j: (0, (b * (N // QB2) + i) * 2 +
                             jnp.maximum(j - 1, 0))),
        out_shape=jax.ShapeDtypeStruct((K, BN), jnp.int32),
        scratch_shapes=[pltpu.VMEM((N, QB2), jnp.float32),
                        pltpu.VMEM((NCH * K, QBLK), jnp.float32),
                        pltpu.VMEM((NCH * K, QBLK), jnp.int32),
                        pltpu.VMEM((N // 16, QB2), jnp.float32)],
    )(h, jnp.swapaxes(h, 1, 2))


PBLK = 512


def _pre_body(h_ref, rf_ref, p_ref, w1hi_ref, w1hj_ref, w1p_ref, b1_ref,
              prej_ref, base_ref, g_ref):
    h = h_ref[...]
    rf = rf_ref[...]
    pp = p_ref[...]
    w1p = w1p_ref[...]
    prej_ref[...] = lax.dot_general(h, w1hj_ref[...], (((1,), (0,)), ((), ())),
                                    preferred_element_type=jnp.float32)
    prei = lax.dot_general(h, w1hi_ref[...], (((1,), (0,)), ((), ())),
                           preferred_element_type=jnp.float32)
    gs = []
    pterm = jnp.zeros((PBLK, D), jnp.float32)
    for c in range(3):
        g_c = (rf[:, 3 * c + 0:3 * c + 1] * w1p[0:1, :]
               + rf[:, 3 * c + 1:3 * c + 2] * w1p[1:2, :]
               + rf[:, 3 * c + 2:3 * c + 3] * w1p[2:3, :])
        gs.append(g_c)
        pterm = pterm + pp[:, c:c + 1] * g_c
    g_ref[...] = jnp.concatenate(gs, axis=1)
    base_ref[...] = prei + b1_ref[...] - pterm


def _precompute(hf, rf, pf, w1hi, w1hj, w1p, b1):
    return pl.pallas_call(
        _pre_body,
        grid=(BN // PBLK,),
        in_specs=[
            pl.BlockSpec((PBLK, D), lambda i: (i, 0)),
            pl.BlockSpec((PBLK, 16), lambda i: (i, 0)),
            pl.BlockSpec((PBLK, 8), lambda i: (i, 0)),
            pl.BlockSpec((D, D), lambda i: (0, 0)),
            pl.BlockSpec((D, D), lambda i: (0, 0)),
            pl.BlockSpec((8, D), lambda i: (0, 0)),
            pl.BlockSpec((1, D), lambda i: (0, 0)),
        ],
        out_specs=[
            pl.BlockSpec((PBLK, D), lambda i: (i, 0)),
            pl.BlockSpec((PBLK, D), lambda i: (i, 0)),
            pl.BlockSpec((PBLK, 3 * D), lambda i: (i, 0)),
        ],
        out_shape=[
            jax.ShapeDtypeStruct((BN, D), jnp.float32),
            jax.ShapeDtypeStruct((BN, D), jnp.float32),
            jax.ShapeDtypeStruct((BN, 3 * D), jnp.float32),
        ],
    )(hf, rf, pf, w1hi, w1hj, w1p, b1)


CQ = 128
CT = CQ * K


def _mlp_body(prej_ref, pj0_ref, pj1_ref, pj2_ref, base_ref, g_ref, w2_ref,
              b2_ref, w3_ref, b3_ref, out_ref):
    prej = prej_ref[...]
    base = base_ref[...]
    g = g_ref[...]
    t = prej.reshape(CQ, K, D) + base[:, None, :]
    for c, pj_ref in enumerate((pj0_ref, pj1_ref, pj2_ref)):
        t = t + pj_ref[...][:, :, None] * g[:, c * D:(c + 1) * D][:, None, :]
    x = jnp.where(t > 0, t, 0.1 * t).reshape(CT, D)
    x = lax.dot_general(x, w2_ref[...], (((1,), (0,)), ((), ())),
                        preferred_element_type=jnp.float32) + b2_ref[...]
    x = jnp.where(x > 0, x, 0.1 * x)
    x = lax.dot_general(x, w3_ref[...], (((1,), (0,)), ((), ())),
                        preferred_element_type=jnp.float32) + b3_ref[...]
    out_ref[...] = jnp.max(x.reshape(CQ, K, D), axis=1)


def _mlp_call(prej_g, pj0, pj1, pj2, base, g, w2, b2, w3, b3):
    return pl.pallas_call(
        _mlp_body,
        grid=(BN // CQ,),
        in_specs=[
            pl.BlockSpec((CT, D), lambda i: (i, 0)),
            pl.BlockSpec((CQ, K), lambda i: (i, 0)),
            pl.BlockSpec((CQ, K), lambda i: (i, 0)),
            pl.BlockSpec((CQ, K), lambda i: (i, 0)),
            pl.BlockSpec((CQ, D), lambda i: (i, 0)),
            pl.BlockSpec((CQ, 3 * D), lambda i: (i, 0)),
            pl.BlockSpec((D, D), lambda i: (0, 0)),
            pl.BlockSpec((1, D), lambda i: (0, 0)),
            pl.BlockSpec((D, D), lambda i: (0, 0)),
            pl.BlockSpec((1, D), lambda i: (0, 0)),
        ],
        out_specs=pl.BlockSpec((CQ, D), lambda i: (i, 0)),
        out_shape=jax.ShapeDtypeStruct((BN, D), jnp.float32),
    )(prej_g, pj0, pj1, pj2, base, g, w2, b2, w3, b3)


NWORK = 32
GW = TOK // NWORK
GC = 128
NWIN = GW // GC


def _gather(prej_table, ptab_flat, idx_flat):
    mesh = plsc.VectorSubcoreMesh(core_axis_name="c", subcore_axis_name="s")

    @functools.partial(
        pl.kernel, mesh=mesh,
        out_type=[jax.ShapeDtypeStruct((TOK, D), jnp.float32),
                  jax.ShapeDtypeStruct((3 * TOK,), jnp.float32)],
        scratch_types=[pltpu.VMEM((GW,), jnp.int32),
                       pltpu.VMEM((GC, D), jnp.float32),
                       pltpu.VMEM((GC, D), jnp.float32),
                       pltpu.VMEM((3 * GC,), jnp.int32),
                       pltpu.VMEM((3 * GC,), jnp.int32),
                       pltpu.VMEM((3 * GC,), jnp.float32),
                       pltpu.VMEM((3 * GC,), jnp.float32),
                       pltpu.SemaphoreType.DMA,
                       pltpu.SemaphoreType.DMA,
                       pltpu.SemaphoreType.DMA,
                       pltpu.SemaphoreType.DMA],
    )
    def k(prej_hbm, ptab_hbm, idx_hbm, o1_hbm, o2_hbm,
          idx_v, rows_a, rows_b, offs_a, offs_b, pv_a, pv_b, sa, sb, pa, pb):
        wid = lax.axis_index("s") * 2 + lax.axis_index("c")
        base = wid * GW
        pltpu.sync_copy(idx_hbm.at[pl.ds(base, GW)], idx_v)

        def fill_offs(w, offs):
            for g in range(GC // 16):
                iv = idx_v[pl.ds(w * GC + g * 16, 16)]
                for c in range(3):
                    offs[pl.ds(c * GC + g * 16, 16)] = iv * 4 + c

        def start_win(w, offs, rows, pv, sr, sp):
            fill_offs(w, offs)
            hr = pltpu.async_copy(
                prej_hbm.at[idx_v.at[pl.ds(w * GC, GC)]], rows, sr)
            hp = pltpu.async_copy(ptab_hbm.at[offs], pv, sp)
            return hr, hp

        def finish_win(w, rows, pv, hr, hp):
            hr.wait()
            pltpu.sync_copy(rows, o1_hbm.at[pl.ds(base + w * GC, GC)])
            hp.wait()
            for c in range(3):
                pltpu.sync_copy(
                    pv.at[pl.ds(c * GC, GC)],
                    o2_hbm.at[pl.ds(c * TOK + base + w * GC, GC)])

        def pair(g, _):
            wa = 2 * g
            wb = 2 * g + 1
            ha = start_win(wa, offs_a, rows_a, pv_a, sa, pa)
            hb = start_win(wb, offs_b, rows_b, pv_b, sb, pb)
            finish_win(wa, rows_a, pv_a, *ha)
            finish_win(wb, rows_b, pv_b, *hb)
            return 0

        lax.fori_loop(0, NWIN // 2, pair, 0)

    return k(prej_table, ptab_flat, idx_flat)


@jax.jit
def kernel(p, R, h, W1, b1, W2, b2, W3, b3):
    hf = h.reshape(BN, D)
    rf = jnp.pad(R.reshape(BN, 9), ((0, 0), (0, 7)))
    pf = jnp.pad(p.reshape(BN, 3), ((0, 0), (0, 5)))
    w1p = jnp.pad(W1[0:3], ((0, 5), (0, 0)))
    w1hi = W1[3:3 + D]
    w1hj = W1[3 + D:3 + 2 * D]
    b1r = b1.reshape(1, D)

    idx = _topk(h).T
    prej_table, base, g = _precompute(hf, rf, pf, w1hi, w1hj, w1p, b1r)
    ptab = jnp.pad(p.reshape(BN, 3), ((0, 0), (0, 1))).reshape(BN * 4)

    prej_g, pj_flat = _gather(prej_table, ptab, idx.reshape(TOK))
    pj3 = pj_flat.reshape(3, BN, K)
    out = _mlp_call(prej_g, pj3[0], pj3[1], pj3[2], base, g, W2,
                    b2.reshape(1, D), W3, b3.reshape(1, D))
    return out.reshape(B, N, D)


# speedup vs baseline: 1.1649x; 1.0067x over previous
import functools
import jax
import jax.numpy as jnp
from jax import lax
from jax.experimental import pallas as pl
from jax.experimental.pallas import tpu as pltpu
from jax.experimental.pallas import tpu_sc as plsc

B, N, D, K = 4, 2048, 128, 24
BN = B * N
TOK = BN * K

BIG_F = 3.0e38
BIG_I = 2**30

QBLK = 128
QB2 = 256
CHR = 256
NCH = N // CHR


KH = 10


def _topk_body(h_all_ref, hq_ref, idx_ref, score_ref, cv_ref, ci_ref, m16_ref):
    b = pl.program_id(0)
    j = pl.program_id(2)

    @pl.when(j == 0)
    def _scores():
        h_all = h_all_ref[0]
        sq = jnp.sum(h_all * h_all, axis=1, keepdims=True) * 0.5
        dot = lax.dot_general(h_all, hq_ref[0], (((1,), (0,)), ((), ())),
                              preferred_element_type=jnp.float32)
        score = sq - dot
        score_ref[...] = score
        m16_ref[...] = jnp.min(score.reshape(N // 16, 16, QB2), axis=1)

    def _extract(hh):
        q0 = hh * QBLK
        kiota = lax.broadcasted_iota(jnp.int32, (K, QBLK), 0)
        riota = lax.broadcasted_iota(jnp.int32, (CHR, QBLK), 0)

        def ext_tau(k, carry):
            mv16, _ = carry
            m = jnp.min(mv16, axis=0, keepdims=True)
            mv16 = jnp.where(mv16 == m, BIG_F, mv16)
            return mv16, m

        _, tau = lax.fori_loop(
            0, K, ext_tau, (m16_ref[:, q0:q0 + QBLK],
                            jnp.zeros((1, QBLK), jnp.float32)))

        def ext_local(k, carry):
            ch, mv, mi = carry
            m = jnp.min(ch, axis=0, keepdims=True)
            key = jnp.where(ch == m, riota, BIG_I)
            am = jnp.min(key, axis=0, keepdims=True)
            ch = jnp.where(key == am, BIG_F, ch)
            mv = jnp.where(kiota == k, m, mv)
            mi = jnp.where(kiota == k, am, mi)
            return ch, mv, mi

        for c in range(NCH):
            ch0 = score_ref[c * CHR:(c + 1) * CHR, q0:q0 + QBLK]
            ch1, mv, mi = lax.fori_loop(
                0, KH, ext_local,
                (ch0, jnp.full((K, QBLK), BIG_F, jnp.float32),
                 jnp.zeros((K, QBLK), jnp.int32)))
            cv_ref[c * K:(c + 1) * K, :] = mv
            ci_ref[c * K:(c + 1) * K, :] = mi + c * CHR
            mnext = jnp.min(ch1, axis=0, keepdims=True)
            need_more = jnp.any(mnext <= tau)

            @pl.when(need_more)
            def _finish(ch1=ch1, mv=mv, mi=mi, c=c):
                _, mv2, mi2 = lax.fori_loop(KH, K, ext_local, (ch1, mv, mi))
                cv_ref[c * K:(c + 1) * K, :] = mv2
                ci_ref[c * K:(c + 1) * K, :] = mi2 + c * CHR

        v0 = cv_ref[...]
        ix = ci_ref[...]

        def ext_merge(k, carry):
            v, cols = carry
            m = jnp.min(v, axis=0, keepdims=True)
            key = jnp.where(v == m, ix, BIG_I)
            am = jnp.min(key, axis=0, keepdims=True)
            v = jnp.where(key == am, BIG_F, v)
            cols = jnp.where(kiota == k, am, cols)
            return v, cols

        _, cols = lax.fori_loop(
            0, K, ext_merge, (v0, jnp.zeros((K, QBLK), jnp.int32)))
        idx_ref[...] = cols + b * N

    for _hh in range(2):
        @pl.when(j == 1 + _hh)
        def _run(_hh=_hh):
            _extract(_hh)


def _topk(h):
    return pl.pallas_call(
        _topk_body,
        grid=(B, N // QB2, 3),
        in_specs=[
            pl.BlockSpec((1, N, D), lambda b, i, j: (b, 0, 0)),
            pl.BlockSpec((1, D, QB2), lambda b, i, j: (b, 0, i)),
        ],
        out_specs=pl.BlockSpec(
            (K, QBLK),
            lambda b, i, j: (0, (b * (N // QB2) + i) * 2 +
                             jnp.maximum(j - 1, 0))),
        out_shape=jax.ShapeDtypeStruct((K, BN), jnp.int32),
        scratch_shapes=[pltpu.VMEM((N, QB2), jnp.float32),
                        pltpu.VMEM((NCH * K, QBLK), jnp.float32),
                        pltpu.VMEM((NCH * K, QBLK), jnp.int32),
                        pltpu.VMEM((N // 16, QB2), jnp.float32)],
    )(h, jnp.swapaxes(h, 1, 2))


PBLK = 512


def _pre_body(h_ref, rf_ref, p_ref, w1hi_ref, w1hj_ref, w1p_ref, b1_ref,
              prej_ref, base_ref, g_ref):
    h = h_ref[...]
    rf = rf_ref[...]
    pp = p_ref[...]
    w1p = w1p_ref[...]
    prej_ref[...] = lax.dot_general(h, w1hj_ref[...], (((1,), (0,)), ((), ())),
                                    preferred_element_type=jnp.float32)
    prei = lax.dot_general(h, w1hi_ref[...], (((1,), (0,)), ((), ())),
                           preferred_element_type=jnp.float32)
    gs = []
    pterm = jnp.zeros((PBLK, D), jnp.float32)
    for c in range(3):
        g_c = (rf[:, 3 * c + 0:3 * c + 1] * w1p[0:1, :]
               + rf[:, 3 * c + 1:3 * c + 2] * w1p[1:2, :]
               + rf[:, 3 * c + 2:3 * c + 3] * w1p[2:3, :])
        gs.append(g_c)
        pterm = pterm + pp[:, c:c + 1] * g_c
    g_ref[...] = jnp.concatenate(gs, axis=1)
    base_ref[...] = prei + b1_ref[...] - pterm


def _precompute(hf, rf, pf, w1hi, w1hj, w1p, b1):
    return pl.pallas_call(
        _pre_body,
        grid=(BN // PBLK,),
        in_specs=[
            pl.BlockSpec((PBLK, D), lambda i: (i, 0)),
            pl.BlockSpec((PBLK, 16), lambda i: (i, 0)),
            pl.BlockSpec((PBLK, 8), lambda i: (i, 0)),
            pl.BlockSpec((D, D), lambda i: (0, 0)),
            pl.BlockSpec((D, D), lambda i: (0, 0)),
            pl.BlockSpec((8, D), lambda i: (0, 0)),
            pl.BlockSpec((1, D), lambda i: (0, 0)),
        ],
        out_specs=[
            pl.BlockSpec((PBLK, D), lambda i: (i, 0)),
            pl.BlockSpec((PBLK, D), lambda i: (i, 0)),
            pl.BlockSpec((PBLK, 3 * D), lambda i: (i, 0)),
        ],
        out_shape=[
            jax.ShapeDtypeStruct((BN, D), jnp.float32),
            jax.ShapeDtypeStruct((BN, D), jnp.float32),
            jax.ShapeDtypeStruct((BN, 3 * D), jnp.float32),
        ],
    )(hf, rf, pf, w1hi, w1hj, w1p, b1)


CQ = 128
CT = CQ * K


def _mlp_body(prej_ref, pj0_ref, pj1_ref, pj2_ref, base_ref, g_ref, w2_ref,
              b2_ref, w3_ref, b3_ref, out_ref):
    prej = prej_ref[...]
    base = base_ref[...]
    g = g_ref[...]
    t = prej.reshape(CQ, K, D) + base[:, None, :]
    for c, pj_ref in enumerate((pj0_ref, pj1_ref, pj2_ref)):
        t = t + pj_ref[...][:, :, None] * g[:, c * D:(c + 1) * D][:, None, :]
    x = jnp.where(t > 0, t, 0.1 * t).reshape(CT, D)
    x = lax.dot_general(x, w2_ref[...], (((1,), (0,)), ((), ())),
                        preferred_element_type=jnp.float32) + b2_ref[...]
    x = jnp.where(x > 0, x, 0.1 * x)
    x = lax.dot_general(x, w3_ref[...], (((1,), (0,)), ((), ())),
                        preferred_element_type=jnp.float32) + b3_ref[...]
    out_ref[...] = jnp.max(x.reshape(CQ, K, D), axis=1)


def _mlp_call(prej_g, pj0, pj1, pj2, base, g, w2, b2, w3, b3):
    return pl.pallas_call(
        _mlp_body,
        grid=(BN // CQ,),
        in_specs=[
            pl.BlockSpec((CT, D), lambda i: (i, 0)),
            pl.BlockSpec((CQ, K), lambda i: (i, 0)),
            pl.BlockSpec((CQ, K), lambda i: (i, 0)),
            pl.BlockSpec((CQ, K), lambda i: (i, 0)),
            pl.BlockSpec((CQ, D), lambda i: (i, 0)),
            pl.BlockSpec((CQ, 3 * D), lambda i: (i, 0)),
            pl.BlockSpec((D, D), lambda i: (0, 0)),
            pl.BlockSpec((1, D), lambda i: (0, 0)),
            pl.BlockSpec((D, D), lambda i: (0, 0)),
            pl.BlockSpec((1, D), lambda i: (0, 0)),
        ],
        out_specs=pl.BlockSpec((CQ, D), lambda i: (i, 0)),
        out_shape=jax.ShapeDtypeStruct((BN, D), jnp.float32),
    )(prej_g, pj0, pj1, pj2, base, g, w2, b2, w3, b3)


NWORK = 32
GW = TOK // NWORK
GC = 128
NWIN = GW // GC


def _gather(prej_table, ptab_flat, idx_flat):
    mesh = plsc.VectorSubcoreMesh(core_axis_name="c", subcore_axis_name="s")

    @functools.partial(
        pl.kernel, mesh=mesh,
        out_type=[jax.ShapeDtypeStruct((TOK, D), jnp.float32),
                  jax.ShapeDtypeStruct((3 * TOK,), jnp.float32)],
        scratch_types=[pltpu.VMEM((GW,), jnp.int32),
                       pltpu.VMEM((GC, D), jnp.float32),
                       pltpu.VMEM((GC, D), jnp.float32),
                       pltpu.VMEM((3 * GC,), jnp.int32),
                       pltpu.VMEM((3 * GC,), jnp.int32),
                       pltpu.VMEM((3 * GW,), jnp.float32),
                       pltpu.SemaphoreType.DMA,
                       pltpu.SemaphoreType.DMA,
                       pltpu.SemaphoreType.DMA,
                       pltpu.SemaphoreType.DMA],
    )
    def k(prej_hbm, ptab_hbm, idx_hbm, o1_hbm, o2_hbm,
          idx_v, rows_a, rows_b, offs_a, offs_b, pv_acc, sa, sb, pa, pb):
        wid = lax.axis_index("s") * 2 + lax.axis_index("c")
        base = wid * GW
        pltpu.sync_copy(idx_hbm.at[pl.ds(base, GW)], idx_v)

        def fill_offs(w, offs):
            for g in range(GC // 16):
                iv = idx_v[pl.ds(w * GC + g * 16, 16)]
                for c in range(3):
                    offs[pl.ds(c * GC + g * 16, 16)] = iv * 4 + c

        def start_win(w, offs, rows, sr, sp):
            fill_offs(w, offs)
            hr = pltpu.async_copy(
                prej_hbm.at[idx_v.at[pl.ds(w * GC, GC)]], rows, sr)
            hp = pltpu.async_copy(
                ptab_hbm.at[offs], pv_acc.at[pl.ds(w * 3 * GC, 3 * GC)], sp)
            return hr, hp

        def finish_win(w, rows, hr, hp):
            hr.wait()
            pltpu.sync_copy(rows, o1_hbm.at[pl.ds(base + w * GC, GC)])
            hp.wait()

        def pair(g, _):
            wa = 2 * g
            wb = 2 * g + 1
            ha = start_win(wa, offs_a, rows_a, sa, pa)
            hb = start_win(wb, offs_b, rows_b, sb, pb)
            finish_win(wa, rows_a, *ha)
            finish_win(wb, rows_b, *hb)
            return 0

        lax.fori_loop(0, NWIN // 2, pair, 0)
        pltpu.sync_copy(pv_acc, o2_hbm.at[pl.ds(base * 3, 3 * GW)])

    return k(prej_table, ptab_flat, idx_flat)


@jax.jit
def kernel(p, R, h, W1, b1, W2, b2, W3, b3):
    hf = h.reshape(BN, D)
    rf = jnp.pad(R.reshape(BN, 9), ((0, 0), (0, 7)))
    pf = jnp.pad(p.reshape(BN, 3), ((0, 0), (0, 5)))
    w1p = jnp.pad(W1[0:3], ((0, 5), (0, 0)))
    w1hi = W1[3:3 + D]
    w1hj = W1[3 + D:3 + 2 * D]
    b1r = b1.reshape(1, D)

    idx = _topk(h).T
    prej_table, base, g = _precompute(hf, rf, pf, w1hi, w1hj, w1p, b1r)
    ptab = jnp.pad(p.reshape(BN, 3), ((0, 0), (0, 1))).reshape(BN * 4)

    prej_g, pj_flat = _gather(prej_table, ptab, idx.reshape(TOK))
    pj3 = pj_flat.reshape(NWORK, NWIN, 3, GC).transpose(2, 0, 1, 3)
    pj3 = pj3.reshape(3, BN, K)
    out = _mlp_call(prej_g, pj3[0], pj3[1], pj3[2], base, g, W2,
                    b2.reshape(1, D), W3, b3.reshape(1, D))
    return out.reshape(B, N, D)
